# baseline (speedup 1.0000x reference)
import functools

import jax
import jax.numpy as jnp
from jax import lax
from jax.experimental import pallas as pl
from jax.experimental.pallas import tpu as pltpu
from jax.experimental.pallas import tpu_sc as plsc

VOCAB = 100000
DIM = 32
B = 4096
L = 50
N_FIELDS = 4

_TOTAL = B * L
_G = 128
_NW = 32
_CPW = _TOTAL // (_NW * _G)
_NPAIR = _CPW // 2


def _make_kernel():
  mesh = plsc.VectorSubcoreMesh(core_axis_name="c", subcore_axis_name="s")

  @functools.partial(
      pl.kernel,
      mesh=mesh,
      compiler_params=pltpu.CompilerParams(use_tc_tiling_on_sc=False),
      out_type=jax.ShapeDtypeStruct((_TOTAL, N_FIELDS * DIM), jnp.float32),
      scratch_types=[
          pltpu.VMEM((N_FIELDS, _CPW, _G), jnp.int32),
          pltpu.VMEM((N_FIELDS, _G, DIM), jnp.float32),
          pltpu.VMEM((N_FIELDS, _G, DIM), jnp.float32),
          pltpu.SemaphoreType.DMA,
          pltpu.SemaphoreType.DMA,
          pltpu.SemaphoreType.DMA,
          pltpu.SemaphoreType.DMA,
      ],
  )
  def k(idx0, idx1, idx2, idx3, t0, t1, t2, t3, out,
        idx_v, buf0, buf1, gs0, gs1, ws0, ws1):
    wid = lax.axis_index("s") * 2 + lax.axis_index("c")
    base = wid * (_CPW * _G)
    tabs = (t0, t1, t2, t3)
    for f, idx_hbm in enumerate((idx0, idx1, idx2, idx3)):
      pltpu.sync_copy(idx_hbm.at[wid], idx_v.at[f])

    def g_copies(c, buf, sem):
      return [pltpu.make_async_copy(tabs[f].at[idx_v.at[f, c]], buf.at[f], sem)
              for f in range(N_FIELDS)]

    def w_copies(c, buf, sem):
      row0 = pl.multiple_of(base + c * _G, 8)
      return [pltpu.make_async_copy(
                  buf.at[f],
                  out.at[pl.ds(row0, _G), pl.ds(f * DIM, DIM)],
                  sem)
              for f in range(N_FIELDS)]

    def fire(copies):
      for cp in copies:
        cp.start()

    def drain(copies):
      for cp in copies:
        cp.wait()

    fire(g_copies(0, buf0, gs0))

    def body(c2, carry):
      a = c2 * 2
      b = a + 1
      drain(g_copies(a, buf0, gs0))

      @pl.when(c2 > 0)
      def _():
        drain(w_copies(b - 2, buf1, ws1))

      fire(g_copies(b, buf1, gs1))
      fire(w_copies(a, buf0, ws0))
      drain(g_copies(b, buf1, gs1))
      drain(w_copies(a, buf0, ws0))

      @pl.when(c2 < _NPAIR - 1)
      def _():
        fire(g_copies(a + 2, buf0, gs0))

      fire(w_copies(b, buf1, ws1))
      return carry

    lax.fori_loop(0, _NPAIR, body, 0)
    drain(w_copies(_CPW - 1, buf1, ws1))

  return k


_sc_kernel = _make_kernel()


def kernel(idx_cat0, idx_cat1, idx_cat2, idx_cat3,
           emb_cat0, emb_cat1, emb_cat2, emb_cat3):
  idxs = [jnp.transpose(i.astype(jnp.int32)).reshape(_NW, _CPW, _G)
          for i in (idx_cat0, idx_cat1, idx_cat2, idx_cat3)]
  out = _sc_kernel(idxs[0], idxs[1], idxs[2], idxs[3],
                   emb_cat0, emb_cat1, emb_cat2, emb_cat3)
  return out.reshape(L, B, N_FIELDS * DIM).transpose(1, 0, 2)


# speedup vs baseline: 1.0697x; 1.0697x over previous
import functools

import jax
import jax.numpy as jnp
from jax import lax
from jax.experimental import pallas as pl
from jax.experimental.pallas import tpu as pltpu
from jax.experimental.pallas import tpu_sc as plsc

VOCAB = 100000
DIM = 32
B = 4096
L = 50
N_FIELDS = 4

_TOTAL = B * L
_G = 128
_NW = 32
_CPW = _TOTAL // (_NW * _G)
_NPAIR = _CPW // 2


def _make_kernel():
  mesh = plsc.VectorSubcoreMesh(core_axis_name="c", subcore_axis_name="s")

  @functools.partial(
      pl.kernel,
      mesh=mesh,
      compiler_params=pltpu.CompilerParams(use_tc_tiling_on_sc=False),
      out_type=jax.ShapeDtypeStruct((_TOTAL, N_FIELDS * DIM), jnp.float32),
      scratch_types=[
          pltpu.VMEM((N_FIELDS, _CPW, _G), jnp.int32),
          pltpu.VMEM((N_FIELDS, _G, DIM), jnp.float32),
          pltpu.VMEM((N_FIELDS, _G, DIM), jnp.float32),
          pltpu.SemaphoreType.DMA,
          pltpu.SemaphoreType.DMA,
          pltpu.SemaphoreType.DMA,
          pltpu.SemaphoreType.DMA,
      ],
  )
  def k(idx0, idx1, idx2, idx3, t0, t1, t2, t3, out,
        idx_v, buf0, buf1, gs0, gs1, ws0, ws1):
    wid = lax.axis_index("s") * 2 + lax.axis_index("c")
    base = wid * (_CPW * _G)
    tabs = (t0, t1, t2, t3)
    for f, idx_hbm in enumerate((idx0, idx1, idx2, idx3)):
      pltpu.sync_copy(idx_hbm.at[wid], idx_v.at[f])

    def g_copies(c, buf, sem):
      return [pltpu.make_async_copy(tabs[f].at[idx_v.at[f, c]], buf.at[f], sem)
              for f in range(N_FIELDS)]

    def w_copies(c, buf, sem):
      row0 = pl.multiple_of(base + c * _G, 8)
      return [pltpu.make_async_copy(
                  buf.at[f],
                  out.at[pl.ds(row0, _G), pl.ds(f * DIM, DIM)],
                  sem)
              for f in range(N_FIELDS)]

    def fire(copies):
      for cp in copies:
        cp.start()

    def drain(copies):
      for cp in copies:
        cp.wait()

    fire(g_copies(0, buf0, gs0))

    def body(c2, carry):
      a = c2 * 2
      b = a + 1
      drain(g_copies(a, buf0, gs0))

      @pl.when(c2 > 0)
      def _():
        drain(w_copies(b - 2, buf1, ws1))

      fire(g_copies(b, buf1, gs1))
      fire(w_copies(a, buf0, ws0))
      drain(w_copies(a, buf0, ws0))

      @pl.when(c2 < _NPAIR - 1)
      def _():
        fire(g_copies(a + 2, buf0, gs0))

      drain(g_copies(b, buf1, gs1))
      fire(w_copies(b, buf1, ws1))
      return carry

    lax.fori_loop(0, _NPAIR, body, 0)
    drain(w_copies(_CPW - 1, buf1, ws1))

  return k


_sc_kernel = _make_kernel()


def kernel(idx_cat0, idx_cat1, idx_cat2, idx_cat3,
           emb_cat0, emb_cat1, emb_cat2, emb_cat3):
  idxs = [jnp.transpose(i.astype(jnp.int32)).reshape(_NW, _CPW, _G)
          for i in (idx_cat0, idx_cat1, idx_cat2, idx_cat3)]
  out = _sc_kernel(idxs[0], idxs[1], idxs[2], idxs[3],
                   emb_cat0, emb_cat1, emb_cat2, emb_cat3)
  return out.reshape(L, B, N_FIELDS * DIM).transpose(1, 0, 2)


# speedup vs baseline: 1.0795x; 1.0092x over previous
import functools

import jax
import jax.numpy as jnp
from jax import lax
from jax.experimental import pallas as pl
from jax.experimental.pallas import tpu as pltpu
from jax.experimental.pallas import tpu_sc as plsc

VOCAB = 100000
DIM = 32
B = 4096
L = 50
N_FIELDS = 4

_TOTAL = B * L
_G = 128
_NW = 32
_CPW = _TOTAL // (_NW * _G)
_NPAIR = _CPW // 2


def _make_kernel():
  mesh = plsc.VectorSubcoreMesh(core_axis_name="c", subcore_axis_name="s")

  @functools.partial(
      pl.kernel,
      mesh=mesh,
      compiler_params=pltpu.CompilerParams(use_tc_tiling_on_sc=False),
      out_type=jax.ShapeDtypeStruct((_TOTAL, N_FIELDS * DIM), jnp.float32),
      scratch_types=[
          pltpu.VMEM((N_FIELDS, _CPW, _G), jnp.int32),
          pltpu.VMEM((N_FIELDS, _G, DIM), jnp.float32),
          pltpu.VMEM((N_FIELDS, _G, DIM), jnp.float32),
          pltpu.VMEM((N_FIELDS, _G, DIM), jnp.float32),
          pltpu.SemaphoreType.DMA,
          pltpu.SemaphoreType.DMA,
          pltpu.SemaphoreType.DMA,
          pltpu.SemaphoreType.DMA,
          pltpu.SemaphoreType.DMA,
          pltpu.SemaphoreType.DMA,
      ],
  )
  def k(idx0, idx1, idx2, idx3, t0, t1, t2, t3, out,
        idx_v, buf0, buf1, buf2, gs0, gs1, gs2, ws0, ws1, ws2):
    wid = lax.axis_index("s") * 2 + lax.axis_index("c")
    base = wid * (_CPW * _G)
    tabs = (t0, t1, t2, t3)
    for f, idx_hbm in enumerate((idx0, idx1, idx2, idx3)):
      pltpu.sync_copy(idx_hbm.at[wid], idx_v.at[f])

    def g_copies(c, buf, sem):
      return [pltpu.make_async_copy(tabs[f].at[idx_v.at[f, c]], buf.at[f], sem)
              for f in range(N_FIELDS)]

    def w_copies(c, buf, sem):
      row0 = pl.multiple_of(base + c * _G, 8)
      return [pltpu.make_async_copy(
                  buf.at[f],
                  out.at[pl.ds(row0, _G), pl.ds(f * DIM, DIM)],
                  sem)
              for f in range(N_FIELDS)]

    def fire(copies):
      for cp in copies:
        cp.start()

    def drain(copies):
      for cp in copies:
        cp.wait()

    bufs = (buf0, buf1, buf2)
    gss = (gs0, gs1, gs2)
    wss = (ws0, ws1, ws2)

    def slot(n, j, in_loop):
      drain(g_copies(n, bufs[j], gss[j]))
      fire(w_copies(n, bufs[j], wss[j]))
      jm1 = (j + 2) % 3

      def drain_prev():
        drain(w_copies(n - 1, bufs[jm1], wss[jm1]))

      if in_loop:
        pl.when(n >= 1)(drain_prev)
      else:
        drain_prev()
      if in_loop:
        jn = (j + 2) % 3
        fire(g_copies(n + 2, bufs[jn], gss[jn]))

    fire(g_copies(0, buf0, gs0))
    fire(g_copies(1, buf1, gs1))

    def body(i, carry):
      for j in range(3):
        slot(i * 3 + j, j, True)
      return carry

    _NTRI = (_CPW - 2) // 3
    lax.fori_loop(0, _NTRI, body, 0)
    slot(_CPW - 2, (_CPW - 2) % 3, False)
    slot(_CPW - 1, (_CPW - 1) % 3, False)
    j_last = (_CPW - 1) % 3
    drain(w_copies(_CPW - 1, bufs[j_last], wss[j_last]))

  return k


_sc_kernel = _make_kernel()


def kernel(idx_cat0, idx_cat1, idx_cat2, idx_cat3,
           emb_cat0, emb_cat1, emb_cat2, emb_cat3):
  idxs = [jnp.transpose(i.astype(jnp.int32)).reshape(_NW, _CPW, _G)
          for i in (idx_cat0, idx_cat1, idx_cat2, idx_cat3)]
  out = _sc_kernel(idxs[0], idxs[1], idxs[2], idxs[3],
                   emb_cat0, emb_cat1, emb_cat2, emb_cat3)
  return out.reshape(L, B, N_FIELDS * DIM).transpose(1, 0, 2)
